# baseline (speedup 1.0000x reference)
import functools

import jax
import jax.numpy as jnp
from jax import lax
from jax.experimental import pallas as pl
from jax.experimental.pallas import tpu as pltpu
from jax.experimental.pallas import tpu_sc as plsc

NC = 2
NS = 16
NW = NC * NS
LANES = 16


@functools.partial(jax.jit, static_argnames=("b_per_w", "D"))
def _sc_gather(idxA, idxL, idxB, table_pad, b_per_w, D):
    V, Dp = table_pad.shape
    B = NW * b_per_w
    T = 50
    TP = 56
    DA = Dp - 128
    TA = T - T % 8
    n_pairs = b_per_w // 2
    assert n_pairs * 2 == b_per_w and n_pairs >= 2
    mesh = plsc.VectorSubcoreMesh(
        core_axis_name="c", subcore_axis_name="s", num_cores=NC, num_subcores=NS
    )

    @functools.partial(
        pl.kernel,
        out_type=jax.ShapeDtypeStruct((B, T, D), jnp.float32),
        mesh=mesh,
        scratch_types=[
            pltpu.VMEM((T, D), jnp.float32),
            pltpu.VMEM((T, D), jnp.float32),
            pltpu.VMEM((TP, 128), jnp.float32),
            pltpu.VMEM((8, DA), jnp.float32),
            pltpu.VMEM((TA,), jnp.int32),
            pltpu.VMEM((TA,), jnp.int32),
            pltpu.VMEM((8,), jnp.int32),
            pltpu.VMEM((8,), jnp.int32),
            pltpu.VMEM((TP,), jnp.int32),
            pltpu.VMEM((TP,), jnp.int32),
            pltpu.SemaphoreType.DMA,
            pltpu.SemaphoreType.DMA,
            pltpu.SemaphoreType.DMA,
            pltpu.SemaphoreType.DMA,
            pltpu.SemaphoreType.DMA,
            pltpu.SemaphoreType.DMA,
        ],
        compiler_params=pltpu.CompilerParams(needs_layout_passes=False),
    )
    def k(idxA_h, idxL_h, idxB_h, tbl, out_hbm, f0, f1, rb, fl,
          ia0, ia1, il0, il1, ib0, ib1, gA, gB, gL, o0, o1, isem):
        wid = lax.axis_index("s") * NC + lax.axis_index("c")
        base = wid * b_per_w
        tblA = tbl.at[:, pl.ds(0, DA)]
        tblB = tbl.at[:, pl.ds(DA, 128)]
        lane = lax.iota(jnp.int32, LANES)
        n16 = (D - DA) // LANES
        tail_cols = (DA + n16 * LANES) + lane
        tail_mask = lane < ((D - DA) % LANES)
        last_cols = (D // LANES * LANES) + lane
        last_mask = lane < (D % LANES)
        ia = (ia0, ia1)
        il = (il0, il1)
        ib = (ib0, ib1)
        fs = (f0, f1)
        os = (o0, o1)

        def merge_tail(f):
            for t in range(T):
                for c in range(n16):
                    f[t, pl.ds(DA + c * LANES, LANES)] = rb[t, pl.ds(c * LANES, LANES)]
                x = rb[t, pl.ds(n16 * LANES, LANES)]
                t_vec = jnp.full((LANES,), t, jnp.int32)
                plsc.store_scatter(f, [t_vec, tail_cols], x, mask=tail_mask)

        def merge_last(f):
            for r in range(T - TA):
                src = TA - 42 + r
                t = TA + r
                for c in range(DA // LANES):
                    f[t, pl.ds(c * LANES, LANES)] = fl[src, pl.ds(c * LANES, LANES)]

        def load_idx(j, s, sync):
            cps = (
                pltpu.make_async_copy(idxA_h.at[wid].at[j], ia[s], isem),
                pltpu.make_async_copy(idxL_h.at[wid].at[j], il[s], isem),
                pltpu.make_async_copy(idxB_h.at[wid].at[j], ib[s], isem),
            )
            for cp in cps:
                cp.start()
            if sync:
                for cp in cps:
                    cp.wait()

        def wait_idx(j, s):
            pltpu.make_async_copy(idxA_h.at[wid].at[j], ia[s], isem).wait()
            pltpu.make_async_copy(idxL_h.at[wid].at[j], il[s], isem).wait()
            pltpu.make_async_copy(idxB_h.at[wid].at[j], ib[s], isem).wait()

        def gathers(s):
            pltpu.async_copy(tblB.at[ib[s]], rb, gB)
            pltpu.async_copy(tblA.at[il[s]], fl, gL)
            pltpu.async_copy(
                tblA.at[ia[s]], fs[s].at[pl.ds(0, TA), pl.ds(0, DA)], gA
            )

        def finish_plane(jdyn, s):
            f = fs[s]
            pltpu.make_async_copy(tblB.at[ib[s]], rb, gB).wait()
            merge_tail(f)
            pltpu.make_async_copy(tblA.at[il[s]], fl, gL).wait()
            merge_last(f)
            pltpu.make_async_copy(
                tblA.at[ia[s]], f.at[pl.ds(0, TA), pl.ds(0, DA)], gA
            ).wait()
            pltpu.async_copy(f, out_hbm.at[base + jdyn], os[s])

        def wait_out(s):
            pltpu.make_async_copy(fs[s], out_hbm.at[base], os[s]).wait()

        load_idx(0, 0, True)
        load_idx(1, 1, False)
        gathers(0)
        finish_plane(0, 0)
        wait_idx(1, 1)
        jn2 = jnp.minimum(2, b_per_w - 1)
        load_idx(jn2, 0, False)
        gathers(1)
        finish_plane(1, 1)

        def pair(p, carry):
            j0 = 2 * p
            wait_idx(j0, 0)
            pltpu.async_copy(tblB.at[ib[0]], rb, gB)
            pltpu.async_copy(tblA.at[il[0]], fl, gL)
            wait_out(0)
            pltpu.async_copy(
                tblA.at[ia[0]], fs[0].at[pl.ds(0, TA), pl.ds(0, DA)], gA
            )
            jn = j0 + 1
            load_idx(jn, 1, False)
            finish_plane(j0, 0)
            wait_idx(jn, 1)
            pltpu.async_copy(tblB.at[ib[1]], rb, gB)
            pltpu.async_copy(tblA.at[il[1]], fl, gL)
            wait_out(1)
            pltpu.async_copy(
                tblA.at[ia[1]], fs[1].at[pl.ds(0, TA), pl.ds(0, DA)], gA
            )
            jn0 = lax.min(j0 + 2, b_per_w - 1) - lax.rem(lax.min(j0 + 2, b_per_w - 1), 2)
            load_idx(jn0, 0, False)
            finish_plane(jn, 1)
            return carry

        lax.fori_loop(1, n_pairs, pair, 0)

        wait_out(0)
        wait_out(1)
        wait_idx(0, 0)

    return k(idxA, idxL, idxB, table_pad)


def kernel(index, table):
    B, T = index.shape
    V, D = table.shape
    b_per_w = B // NW
    assert b_per_w * NW == B and T == 50
    Dp = (D + 127) // 128 * 128
    if Dp == D:
        Dp = D + 128
    table_pad = jnp.pad(table, ((0, 0), (0, Dp - D)))
    idx = index.astype(jnp.int32)
    TA = T - T % 8
    idxA = idx[:, :TA].reshape(NW, b_per_w, TA)
    idxL = idx[:, T - 8:].reshape(NW, b_per_w, 8)
    idxB = jnp.pad(idx, ((0, 0), (0, 6))).reshape(NW, b_per_w, 56)
    return _sc_gather(idxA, idxL, idxB, table_pad, b_per_w, D)


# speedup vs baseline: 1.9982x; 1.9982x over previous
import functools

import jax
import jax.numpy as jnp
from jax import lax
from jax.experimental import pallas as pl
from jax.experimental.pallas import tpu as pltpu
from jax.experimental.pallas import tpu_sc as plsc

NC = 2
NS = 16
NW = NC * NS
LANES = 16


@functools.partial(jax.jit, static_argnames=("b_per_w", "D"))
def _sc_gather(idxA, idxL, table_pad, b_per_w, D):
    V, Dp = table_pad.shape
    B = NW * b_per_w
    T = 50
    DA = Dp - 128
    TA = T - T % 8
    mesh = plsc.VectorSubcoreMesh(
        core_axis_name="c", subcore_axis_name="s", num_cores=NC, num_subcores=NS
    )

    @functools.partial(
        pl.kernel,
        out_type=jax.ShapeDtypeStruct((B, T, D), jnp.float32),
        mesh=mesh,
        scratch_types=[
            pltpu.VMEM((2, 1, TA), jnp.int32),
            pltpu.VMEM((2, 1, 8), jnp.int32),
            pltpu.VMEM((T, D), jnp.float32),
            pltpu.VMEM((TA, 128), jnp.float32),
            pltpu.VMEM((8, Dp), jnp.float32),
            pltpu.SemaphoreType.DMA,
            pltpu.SemaphoreType.DMA,
            pltpu.SemaphoreType.DMA,
            pltpu.SemaphoreType.DMA,
            pltpu.SemaphoreType.DMA,
        ],
        compiler_params=pltpu.CompilerParams(needs_layout_passes=False),
    )
    def k(idxA_h, idxL_h, tbl, out_hbm, iav, ilv, f, rb, fl,
          gA, gB, gL, o, isem):
        wid = lax.axis_index("s") * NC + lax.axis_index("c")
        base = wid * b_per_w
        tblA = tbl.at[:, pl.ds(0, DA)]
        tblB = tbl.at[:, pl.ds(DA, 128)]
        lane = lax.iota(jnp.int32, LANES)
        n16 = (D - DA) // LANES
        tail_cols = (DA + n16 * LANES) + lane
        tail_mask = lane < ((D - DA) % LANES)
        last_cols = (D // LANES * LANES) + lane
        last_mask = lane < (D % LANES)

        def merge_tail():
            for t in range(TA):
                for c in range(n16):
                    f[t, pl.ds(DA + c * LANES, LANES)] = rb[t, pl.ds(c * LANES, LANES)]
                x = rb[t, pl.ds(n16 * LANES, LANES)]
                t_vec = jnp.full((LANES,), t, jnp.int32)
                plsc.store_scatter(f, [t_vec, tail_cols], x, mask=tail_mask)

        def merge_last():
            for r in range(T - TA):
                src = TA - 42 + r
                t = TA + r
                for c in range(D // LANES):
                    f[t, pl.ds(c * LANES, LANES)] = fl[src, pl.ds(c * LANES, LANES)]
                x = fl[src, pl.ds(D // LANES * LANES, LANES)]
                t_vec = jnp.full((LANES,), t, jnp.int32)
                plsc.store_scatter(f, [t_vec, last_cols], x, mask=last_mask)

        def ia(slot):
            return iav.at[slot].at[0]

        def il(slot):
            return ilv.at[slot].at[0]

        def load_idx(j, slot, sync):
            cps = (
                pltpu.make_async_copy(idxA_h.at[wid].at[j], iav.at[slot], isem),
                pltpu.make_async_copy(idxL_h.at[wid].at[j], ilv.at[slot], isem),
            )
            for cp in cps:
                cp.start()
            if sync:
                for cp in cps:
                    cp.wait()

        def wait_idx(j, slot):
            pltpu.make_async_copy(idxA_h.at[wid].at[j], iav.at[slot], isem).wait()
            pltpu.make_async_copy(idxL_h.at[wid].at[j], ilv.at[slot], isem).wait()

        load_idx(0, 0, True)
        load_idx(1, 1, False)
        pltpu.async_copy(tblB.at[ia(0)], rb, gB)
        pltpu.async_copy(tbl.at[il(0)], fl, gL)
        pltpu.async_copy(tblA.at[ia(0)], f.at[pl.ds(0, TA), pl.ds(0, DA)], gA)
        pltpu.make_async_copy(tblB.at[ia(0)], rb, gB).wait()
        merge_tail()
        pltpu.make_async_copy(tbl.at[il(0)], fl, gL).wait()
        merge_last()
        pltpu.make_async_copy(
            tblA.at[ia(0)], f.at[pl.ds(0, TA), pl.ds(0, DA)], gA
        ).wait()
        pltpu.async_copy(f, out_hbm.at[base], o)

        def plane(j, carry):
            slot = j % 2
            nslot = (j + 1) % 2
            wait_idx(j, slot)
            jn = lax.min(j + 1, b_per_w - 1)
            load_idx(jn, nslot, False)
            pltpu.async_copy(tblB.at[ia(slot)], rb, gB)
            pltpu.async_copy(tbl.at[il(slot)], fl, gL)
            pltpu.make_async_copy(f, out_hbm.at[base], o).wait()
            pltpu.async_copy(tblA.at[ia(slot)], f.at[pl.ds(0, TA), pl.ds(0, DA)], gA)
            pltpu.make_async_copy(tblB.at[ia(slot)], rb, gB).wait()
            merge_tail()
            pltpu.make_async_copy(tbl.at[il(slot)], fl, gL).wait()
            merge_last()
            pltpu.make_async_copy(
                tblA.at[ia(slot)], f.at[pl.ds(0, TA), pl.ds(0, DA)], gA
            ).wait()
            pltpu.async_copy(f, out_hbm.at[base + j], o)
            return carry

        lax.fori_loop(1, b_per_w, plane, 0)

        pltpu.make_async_copy(f, out_hbm.at[base], o).wait()
        wait_idx(0, 0)

    return k(idxA, idxL, table_pad)


def kernel(index, table):
    B, T = index.shape
    V, D = table.shape
    b_per_w = B // NW
    assert b_per_w * NW == B and T == 50
    Dp = (D + 127) // 128 * 128
    if Dp == D:
        Dp = D + 128
    table_pad = jnp.pad(table, ((0, 0), (0, Dp - D)))
    idx = index.astype(jnp.int32)
    TA = T - T % 8
    idxA = idx[:, :TA].reshape(NW, b_per_w, 1, TA)
    idxL = idx[:, T - 8:].reshape(NW, b_per_w, 1, 8)
    return _sc_gather(idxA, idxL, table_pad, b_per_w, D)
